# baseline (speedup 1.0000x reference)
import jax
import jax.numpy as jnp
from jax import lax
from jax.experimental import pallas as pl
from jax.experimental.pallas import tpu as pltpu
from jax.experimental.pallas import tpu_sc as plsc

B, S, V = 8, 64, 32000
ENC = 400
EH, DH2, DI = 512, 1024, 256
PGEN_D = EH + DH2 + DI
OOV = 64
VE = V + OOV
NC, NS = 2, 16
NW = NC * NS
WPB = NW // B
SPW = S // WPB
LANE = 16


def _vsum16(x):
    lane = lax.iota(jnp.int32, LANE)
    for sh in (8, 4, 2, 1):
        idx = jnp.bitwise_xor(lane, sh)
        x = x + x.at[idx].get(mode="promise_in_bounds")
    return x


def _dot_chunks(buf, b0, wv, w0, n, accs):
    def body(j, accs):
        a0, a1, a2, a3 = accs
        base = b0 + j * (4 * LANE)
        wbase = w0 + j * (4 * LANE)
        a0 = a0 + buf[pl.ds(base, LANE)] * wv[pl.ds(wbase, LANE)]
        a1 = a1 + buf[pl.ds(base + LANE, LANE)] * wv[pl.ds(wbase + LANE, LANE)]
        a2 = a2 + buf[pl.ds(base + 2 * LANE, LANE)] * wv[pl.ds(wbase + 2 * LANE, LANE)]
        a3 = a3 + buf[pl.ds(base + 3 * LANE, LANE)] * wv[pl.ds(wbase + 3 * LANE, LANE)]
        return (a0, a1, a2, a3)
    return lax.fori_loop(0, n // 4, body, accs)


def _softmax_scatter_row(row_v, attn16_v, enc_v, pg, pc, i):
    U = 16
    CW = LANE * U

    def pa(j, accs):
        base = j * CW
        a0, a1, a2, a3 = accs
        vs = [jnp.exp(row_v[pl.ds(base + k * LANE, LANE)]) for k in range(U)]
        for k in range(0, U, 4):
            a0 = a0 + vs[k]
            a1 = a1 + vs[k + 1]
            a2 = a2 + vs[k + 2]
            a3 = a3 + vs[k + 3]
        return (a0, a1, a2, a3)

    zero = jnp.zeros((LANE,), jnp.float32)
    accs = lax.fori_loop(0, V // CW, pa, (zero, zero, zero, zero))
    acc = (accs[0] + accs[1]) + (accs[2] + accs[3])
    t = pg / _vsum16(acc)

    def pb(j, c):
        base = j * CW
        for k in range(U):
            sl = pl.ds(base + k * LANE, LANE)
            row_v[sl] = jnp.exp(row_v[sl]) * t
        return c

    lax.fori_loop(0, V // CW, pb, 0)
    for j in range(OOV // LANE):
        row_v[pl.ds(V + j * LANE, LANE)] = zero

    for j in range(ENC // LANE):
        sl = pl.ds(j * LANE, LANE)
        plsc.addupdate_scatter(row_v, [enc_v[sl]], attn16_v[i, sl] * pc)


def _sc_body(logits, attn, ctx, hid, din, enc, wf, bf, out, pgen_out,
             row0_v, row1_v, row2_v, g0_v, g1_v, attn16_v, enc_v, wv, bv,
             pgv, isem0, isem1, isem2, osem0, osem1, osem2, gsem0, gsem1):
    cid = lax.axis_index("c")
    sid = lax.axis_index("s")
    wid = sid * NC + cid
    b = wid // WPB
    s0 = (wid % WPB) * SPW
    pltpu.sync_copy(enc.at[b], enc_v)
    pltpu.sync_copy(wf, wv)
    pltpu.sync_copy(bf, bv)
    pltpu.sync_copy(attn.at[b, pl.ds(s0, SPW)], attn16_v)
    bufs = (row0_v, row1_v, row2_v)
    isems = (isem0, isem1, isem2)
    osems = (osem0, osem1, osem2)
    gbufs = (g0_v, g1_v)
    gsems = (gsem0, gsem1)
    lane = lax.iota(jnp.int32, LANE)

    def in_copy(i):
        return pltpu.make_async_copy(
            logits.at[b, s0 + i], bufs[i % 3].at[pl.ds(0, V)], isems[i % 3])

    def out_copy(i):
        return pltpu.make_async_copy(
            bufs[i % 3], out.at[b, s0 + i], osems[i % 3])

    def gate_copies(i):
        gb = gbufs[i % 2]
        gs = gsems[i % 2]
        return (
            pltpu.make_async_copy(ctx.at[b, s0 + i], gb.at[pl.ds(0, EH)], gs),
            pltpu.make_async_copy(hid.at[b, s0 + i], gb.at[pl.ds(EH, DH2)], gs),
            pltpu.make_async_copy(din.at[b, s0 + i],
                                  gb.at[pl.ds(EH + DH2, DI)], gs),
        )

    in_copy(0).start()
    for c in gate_copies(0):
        c.start()
    bvv = bv[...]
    zero = jnp.zeros((LANE,), jnp.float32)
    pgen_acc = zero

    for i in range(SPW):
        if i >= 2:
            out_copy(i - 2).wait()
        if i + 1 < SPW:
            in_copy(i + 1).start()
            for c in gate_copies(i + 1):
                c.start()
        for c in gate_copies(i):
            c.wait()
        gb = gbufs[i % 2]
        accs = _dot_chunks(gb, 0, wv, 0, PGEN_D // LANE, (zero,) * 4)
        z = _vsum16((accs[0] + accs[1]) + (accs[2] + accs[3])) + bvv
        pg = 1.0 / (1.0 + jnp.exp(-z))
        pc = 1.0 - pg
        pgen_acc = jnp.where(lane == i, pg, pgen_acc)
        in_copy(i).wait()
        _softmax_scatter_row(bufs[i % 3], attn16_v, enc_v, pg, pc, i)
        out_copy(i).start()
    pgv[...] = pgen_acc
    pltpu.sync_copy(pgv, pgen_out.at[b, pl.ds(s0, SPW)])
    for i in range(SPW - 2, SPW):
        out_copy(i).wait()


def kernel(output_logits, attn_weights, decoder_hidden_state, decoder_input,
           context_vector, encoder_input, max_oovs, W_pgen, b_pgen):
    del max_oovs
    enc = encoder_input.astype(jnp.int32)
    wf = W_pgen.reshape(PGEN_D)
    bf = jnp.full((LANE,), b_pgen[0], jnp.float32)
    sc = pl.kernel(
        _sc_body,
        out_type=[
            jax.ShapeDtypeStruct((B, S, VE), jnp.float32),
            jax.ShapeDtypeStruct((B, S), jnp.float32),
        ],
        mesh=plsc.VectorSubcoreMesh(core_axis_name="c", subcore_axis_name="s"),
        compiler_params=pltpu.CompilerParams(needs_layout_passes=False),
        scratch_types=[
            pltpu.VMEM((VE,), jnp.float32),
            pltpu.VMEM((VE,), jnp.float32),
            pltpu.VMEM((VE,), jnp.float32),
            pltpu.VMEM((PGEN_D,), jnp.float32),
            pltpu.VMEM((PGEN_D,), jnp.float32),
            pltpu.VMEM((SPW, ENC), jnp.float32),
            pltpu.VMEM((ENC,), jnp.int32),
            pltpu.VMEM((PGEN_D,), jnp.float32),
            pltpu.VMEM((LANE,), jnp.float32),
            pltpu.VMEM((LANE,), jnp.float32),
            pltpu.SemaphoreType.DMA,
            pltpu.SemaphoreType.DMA,
            pltpu.SemaphoreType.DMA,
            pltpu.SemaphoreType.DMA,
            pltpu.SemaphoreType.DMA,
            pltpu.SemaphoreType.DMA,
            pltpu.SemaphoreType.DMA,
            pltpu.SemaphoreType.DMA,
        ],
    )
    total, pgen2 = sc(output_logits, attn_weights, context_vector,
                      decoder_hidden_state, decoder_input, enc, wf, bf)
    return total, pgen2.reshape(B, S, 1)


# speedup vs baseline: 1.0839x; 1.0839x over previous
import jax
import jax.numpy as jnp
from jax import lax
from jax.experimental import pallas as pl
from jax.experimental.pallas import tpu as pltpu
from jax.experimental.pallas import tpu_sc as plsc

B, S, V = 8, 64, 32000
ENC = 400
EH, DH2, DI = 512, 1024, 256
PGEN_D = EH + DH2 + DI
OOV = 64
VE = V + OOV
NC, NS = 2, 16
NW = NC * NS
WPB = NW // B
SPW = S // WPB
LANE = 16


def _vsum16(x):
    lane = lax.iota(jnp.int32, LANE)
    for sh in (8, 4, 2, 1):
        idx = jnp.bitwise_xor(lane, sh)
        x = x + x.at[idx].get(mode="promise_in_bounds")
    return x


def _dot_chunks(buf, b0, wv, w0, n, accs):
    def body(j, accs):
        a0, a1, a2, a3 = accs
        base = b0 + j * (4 * LANE)
        wbase = w0 + j * (4 * LANE)
        a0 = a0 + buf[pl.ds(base, LANE)] * wv[pl.ds(wbase, LANE)]
        a1 = a1 + buf[pl.ds(base + LANE, LANE)] * wv[pl.ds(wbase + LANE, LANE)]
        a2 = a2 + buf[pl.ds(base + 2 * LANE, LANE)] * wv[pl.ds(wbase + 2 * LANE, LANE)]
        a3 = a3 + buf[pl.ds(base + 3 * LANE, LANE)] * wv[pl.ds(wbase + 3 * LANE, LANE)]
        return (a0, a1, a2, a3)
    return lax.fori_loop(0, n // 4, body, accs)


def _softmax_scatter_row(row_v, attn16_v, enc_v, pg, pc, i):
    U = 16
    CW = LANE * U

    def pa(j, accs):
        base = j * CW
        a0, a1, a2, a3 = accs
        vs = []
        for k in range(U):
            sl = pl.ds(base + k * LANE, LANE)
            v = jnp.exp(row_v[sl])
            row_v[sl] = v
            vs.append(v)
        for k in range(0, U, 4):
            a0 = a0 + vs[k]
            a1 = a1 + vs[k + 1]
            a2 = a2 + vs[k + 2]
            a3 = a3 + vs[k + 3]
        return (a0, a1, a2, a3)

    zero = jnp.zeros((LANE,), jnp.float32)
    accs = lax.fori_loop(0, V // CW, pa, (zero, zero, zero, zero))
    acc = (accs[0] + accs[1]) + (accs[2] + accs[3])
    t = pg / _vsum16(acc)

    def pb(j, c):
        base = j * CW
        for k in range(U):
            sl = pl.ds(base + k * LANE, LANE)
            row_v[sl] = row_v[sl] * t
        return c

    lax.fori_loop(0, V // CW, pb, 0)
    for j in range(OOV // LANE):
        row_v[pl.ds(V + j * LANE, LANE)] = zero

    for j in range(ENC // LANE):
        sl = pl.ds(j * LANE, LANE)
        plsc.addupdate_scatter(row_v, [enc_v[sl]], attn16_v[i, sl] * pc)


def _sc_body(logits, attn, ctx, hid, din, enc, wb, out, pgen_out,
             row0_v, row1_v, row2_v, g0_v, g1_v, attn16_v, enc_v, wv, pgv,
             isems, osems, gsems):
    cid = lax.axis_index("c")
    sid = lax.axis_index("s")
    wid = sid * NC + cid
    b = wid // WPB
    s0 = (wid % WPB) * SPW
    pltpu.sync_copy(enc.at[b], enc_v)
    pltpu.sync_copy(wb, wv)
    pltpu.sync_copy(attn.at[b, pl.ds(s0, SPW)], attn16_v)
    lane = lax.iota(jnp.int32, LANE)
    bufs = (row0_v, row1_v, row2_v)
    gbufs = (g0_v, g1_v)

    def in_copy(i):
        k = i % 3
        return pltpu.make_async_copy(
            logits.at[b, s0 + i], bufs[k].at[pl.ds(0, V)], isems.at[k])

    def out_copy(i):
        k = i % 3
        return pltpu.make_async_copy(
            bufs[k], out.at[b, s0 + i], osems.at[k])

    def gate_copies(i):
        k = i % 2
        gb = gbufs[k]
        gs = gsems.at[k]
        return (
            pltpu.make_async_copy(ctx.at[b, s0 + i], gb.at[pl.ds(0, EH)], gs),
            pltpu.make_async_copy(hid.at[b, s0 + i], gb.at[pl.ds(EH, DH2)], gs),
            pltpu.make_async_copy(din.at[b, s0 + i],
                                  gb.at[pl.ds(EH + DH2, DI)], gs),
        )

    in_copy(0).start()
    for c in gate_copies(0):
        c.start()
    bvv = wv[pl.ds(PGEN_D, LANE)]
    zero = jnp.zeros((LANE,), jnp.float32)
    pgen_acc = zero

    for i in range(SPW):
        if i >= 2:
            out_copy(i - 2).wait()
        if i + 1 < SPW:
            in_copy(i + 1).start()
            for c in gate_copies(i + 1):
                c.start()
        for c in gate_copies(i):
            c.wait()
        accs = _dot_chunks(gbufs[i % 2], 0, wv, 0,
                           PGEN_D // LANE, (zero,) * 4)
        z = _vsum16((accs[0] + accs[1]) + (accs[2] + accs[3])) + bvv
        pg = 1.0 / (1.0 + jnp.exp(-z))
        pc = 1.0 - pg
        pgen_acc = jnp.where(lane == i, pg, pgen_acc)
        in_copy(i).wait()
        _softmax_scatter_row(bufs[i % 3], attn16_v, enc_v, pg, pc, i)
        out_copy(i).start()
    pgv[...] = pgen_acc
    pltpu.sync_copy(pgv, pgen_out.at[b, pl.ds(s0, SPW)])
    for i in range(SPW - 2, SPW):
        out_copy(i).wait()


def kernel(output_logits, attn_weights, decoder_hidden_state, decoder_input,
           context_vector, encoder_input, max_oovs, W_pgen, b_pgen):
    del max_oovs
    enc = encoder_input.astype(jnp.int32)
    wb = jnp.concatenate(
        [W_pgen.reshape(PGEN_D),
         jnp.full((LANE,), b_pgen[0], jnp.float32)])
    sc = pl.kernel(
        _sc_body,
        out_type=[
            jax.ShapeDtypeStruct((B, S, VE), jnp.float32),
            jax.ShapeDtypeStruct((B, S), jnp.float32),
        ],
        mesh=plsc.VectorSubcoreMesh(core_axis_name="c", subcore_axis_name="s"),
        compiler_params=pltpu.CompilerParams(needs_layout_passes=False),
        scratch_types=[
            pltpu.VMEM((VE,), jnp.float32),
            pltpu.VMEM((VE,), jnp.float32),
            pltpu.VMEM((VE,), jnp.float32),
            pltpu.VMEM((PGEN_D,), jnp.float32),
            pltpu.VMEM((PGEN_D,), jnp.float32),
            pltpu.VMEM((SPW, ENC), jnp.float32),
            pltpu.VMEM((ENC,), jnp.int32),
            pltpu.VMEM((PGEN_D + LANE,), jnp.float32),
            pltpu.VMEM((LANE,), jnp.float32),
            pltpu.SemaphoreType.DMA((3,)),
            pltpu.SemaphoreType.DMA((3,)),
            pltpu.SemaphoreType.DMA((2,)),
        ],
    )
    total, pgen2 = sc(output_logits, attn_weights, context_vector,
                      decoder_hidden_state, decoder_input, enc, wb)
    return total, pgen2.reshape(B, S, 1)


# speedup vs baseline: 1.0913x; 1.0068x over previous
import jax
import jax.numpy as jnp
from jax import lax
from jax.experimental import pallas as pl
from jax.experimental.pallas import tpu as pltpu
from jax.experimental.pallas import tpu_sc as plsc

B, S, V = 8, 64, 32000
ENC = 400
PGEN_D = 512 + 1024 + 256
OOV = 64
VE = V + OOV
NC, NS = 2, 16
NW = NC * NS
WPB = NW // B
SPW = S // WPB
LANE = 16


def _gate_body(attn_ref, pre_ref, w_ref, b_ref, pgen_ref, ap_ref):
    pre = pre_ref[...]
    w = w_ref[...]
    z = jnp.sum(pre * w, axis=-1) + b_ref[0, 0]
    pgen = jax.nn.sigmoid(z)
    pgen_ref[...] = pgen
    ap_ref[...] = attn_ref[...] * (1.0 - pgen)[:, :, None]


def _vsum16(x):
    lane = lax.iota(jnp.int32, LANE)
    for sh in (8, 4, 2, 1):
        idx = jnp.bitwise_xor(lane, sh)
        x = x + x.at[idx].get(mode="promise_in_bounds")
    return x


def _compute_row(row_v, attn16_v, enc_v, pg, i):
    U = 16
    CW = LANE * U

    def pa(j, accs):
        base = j * CW
        a0, a1, a2, a3 = accs
        vs = []
        for k in range(U):
            sl = pl.ds(base + k * LANE, LANE)
            v = jnp.exp(row_v[sl])
            row_v[sl] = v
            vs.append(v)
        for k in range(0, U, 4):
            a0 = a0 + vs[k]
            a1 = a1 + vs[k + 1]
            a2 = a2 + vs[k + 2]
            a3 = a3 + vs[k + 3]
        return (a0, a1, a2, a3)

    zero = jnp.zeros((LANE,), jnp.float32)
    accs = lax.fori_loop(0, V // CW, pa, (zero, zero, zero, zero))
    acc = (accs[0] + accs[1]) + (accs[2] + accs[3])
    t = pg / _vsum16(acc)

    def pb(j, c):
        base = j * CW
        for k in range(U):
            sl = pl.ds(base + k * LANE, LANE)
            row_v[sl] = row_v[sl] * t
        return c

    lax.fori_loop(0, V // CW, pb, 0)
    for j in range(OOV // LANE):
        row_v[pl.ds(V + j * LANE, LANE)] = zero

    for j in range(ENC // LANE):
        sl = pl.ds(j * LANE, LANE)
        plsc.addupdate_scatter(row_v, [enc_v[sl]],
                               attn16_v[i, pl.ds(j * LANE, LANE)])


def _sc_body(logits, aprime, pgen2, enc, out,
             row0_v, row1_v, row2_v, attn16_v, enc_v, pgen_v,
             isem0, isem1, isem2, osem0, osem1, osem2):
    cid = lax.axis_index("c")
    sid = lax.axis_index("s")
    wid = sid * NC + cid
    b = wid // WPB
    s0 = (wid % WPB) * SPW
    pltpu.sync_copy(enc.at[b], enc_v)
    pltpu.sync_copy(pgen2.at[b, pl.ds(s0, SPW)], pgen_v)
    pltpu.sync_copy(aprime.at[b, pl.ds(s0, SPW)], attn16_v)
    pv = pgen_v[...]
    bufs = (row0_v, row1_v, row2_v)
    isems = (isem0, isem1, isem2)
    osems = (osem0, osem1, osem2)

    def in_copy(i):
        return pltpu.make_async_copy(
            logits.at[b, s0 + i], bufs[i % 3].at[pl.ds(0, V)], isems[i % 3])

    def out_copy(i):
        return pltpu.make_async_copy(
            bufs[i % 3], out.at[b, s0 + i], osems[i % 3])

    in_copy(0).start()
    for i in range(SPW):
        if i >= 2:
            out_copy(i - 2).wait()
        if i + 1 < SPW:
            in_copy(i + 1).start()
        in_copy(i).wait()
        idx_i = jnp.full((LANE,), i, jnp.int32)
        pg = pv.at[idx_i].get(mode="promise_in_bounds")
        _compute_row(bufs[i % 3], attn16_v, enc_v, pg, i)
        out_copy(i).start()
    for i in range(SPW - 2, SPW):
        out_copy(i).wait()


def kernel(output_logits, attn_weights, decoder_hidden_state, decoder_input,
           context_vector, encoder_input, max_oovs, W_pgen, b_pgen):
    del max_oovs
    pre = jnp.concatenate(
        [context_vector, decoder_hidden_state, decoder_input], axis=-1)
    w3 = W_pgen.reshape(1, 1, PGEN_D)
    b2 = b_pgen.reshape(1, 1)
    pgen2, aprime = pl.pallas_call(
        _gate_body,
        out_shape=[
            jax.ShapeDtypeStruct((B, S), jnp.float32),
            jax.ShapeDtypeStruct((B, S, ENC), jnp.float32),
        ],
    )(attn_weights, pre, w3, b2)

    enc = encoder_input.astype(jnp.int32)
    sc = pl.kernel(
        _sc_body,
        out_type=jax.ShapeDtypeStruct((B, S, VE), jnp.float32),
        mesh=plsc.VectorSubcoreMesh(core_axis_name="c", subcore_axis_name="s"),
        compiler_params=pltpu.CompilerParams(needs_layout_passes=False),
        scratch_types=[
            pltpu.VMEM((VE,), jnp.float32),
            pltpu.VMEM((VE,), jnp.float32),
            pltpu.VMEM((VE,), jnp.float32),
            pltpu.VMEM((SPW, ENC), jnp.float32),
            pltpu.VMEM((ENC,), jnp.int32),
            pltpu.VMEM((SPW,), jnp.float32),
            pltpu.SemaphoreType.DMA,
            pltpu.SemaphoreType.DMA,
            pltpu.SemaphoreType.DMA,
            pltpu.SemaphoreType.DMA,
            pltpu.SemaphoreType.DMA,
            pltpu.SemaphoreType.DMA,
        ],
    )
    total = sc(output_logits, aprime, pgen2, enc)
    return total, pgen2.reshape(B, S, 1)


# speedup vs baseline: 1.1242x; 1.0301x over previous
import jax
import jax.numpy as jnp
from jax import lax
from jax.experimental import pallas as pl
from jax.experimental.pallas import tpu as pltpu
from jax.experimental.pallas import tpu_sc as plsc

B, S, V = 8, 64, 32000
ENC = 400
PGEN_D = 512 + 1024 + 256
OOV = 64
VE = V + OOV
NC, NS = 2, 16
NW = NC * NS
WPB = NW // B
SPW = S // WPB
LANE = 16


EH, DH2, DI = 512, 1024, 256


def _gate_body(attn_ref, ctx_ref, hid_ref, din_ref, w_ref, b_ref,
               pgen_ref, ap_ref):
    w = w_ref[...]
    z = (jnp.sum(ctx_ref[...] * w[:, :, :EH], axis=-1)
         + jnp.sum(hid_ref[...] * w[:, :, EH:EH + DH2], axis=-1)
         + jnp.sum(din_ref[...] * w[:, :, EH + DH2:], axis=-1)
         + b_ref[0, 0])
    pgen = jax.nn.sigmoid(z)
    pgen_ref[...] = pgen
    ap_ref[...] = attn_ref[...] * (1.0 - pgen)[:, :, None]


def _vsum16(x):
    lane = lax.iota(jnp.int32, LANE)
    for sh in (8, 4, 2, 1):
        idx = jnp.bitwise_xor(lane, sh)
        x = x + x.at[idx].get(mode="promise_in_bounds")
    return x


def _compute_row(row_v, attn16_v, enc_v, pg, i):
    U = 16
    CW = LANE * U

    def pa(j, accs):
        base = j * CW
        a0, a1, a2, a3 = accs
        vs = []
        for k in range(U):
            sl = pl.ds(base + k * LANE, LANE)
            v = jnp.exp(row_v[sl])
            row_v[sl] = v
            vs.append(v)
        for k in range(0, U, 4):
            a0 = a0 + vs[k]
            a1 = a1 + vs[k + 1]
            a2 = a2 + vs[k + 2]
            a3 = a3 + vs[k + 3]
        return (a0, a1, a2, a3)

    zero = jnp.zeros((LANE,), jnp.float32)
    accs = lax.fori_loop(0, V // CW, pa, (zero, zero, zero, zero))
    acc = (accs[0] + accs[1]) + (accs[2] + accs[3])
    t = pg / _vsum16(acc)

    def pb(j, c):
        base = j * CW
        for k in range(U):
            sl = pl.ds(base + k * LANE, LANE)
            row_v[sl] = row_v[sl] * t
        return c

    lax.fori_loop(0, V // CW, pb, 0)
    for j in range(OOV // LANE):
        row_v[pl.ds(V + j * LANE, LANE)] = zero

    for j in range(ENC // LANE):
        sl = pl.ds(j * LANE, LANE)
        plsc.addupdate_scatter(row_v, [enc_v[sl]],
                               attn16_v[i, pl.ds(j * LANE, LANE)])


def _sc_body(logits, aprime, pgen2, enc, out,
             row0_v, row1_v, row2_v, attn16_v, enc_v, pgen_v,
             isem0, isem1, isem2, osem0, osem1, osem2):
    cid = lax.axis_index("c")
    sid = lax.axis_index("s")
    wid = sid * NC + cid
    b = wid // WPB
    s0 = (wid % WPB) * SPW
    pltpu.sync_copy(enc.at[b], enc_v)
    pltpu.sync_copy(pgen2.at[b, pl.ds(s0, SPW)], pgen_v)
    pltpu.sync_copy(aprime.at[b, pl.ds(s0, SPW)], attn16_v)
    pv = pgen_v[...]
    bufs = (row0_v, row1_v, row2_v)
    isems = (isem0, isem1, isem2)
    osems = (osem0, osem1, osem2)

    def in_copy(i):
        return pltpu.make_async_copy(
            logits.at[b, s0 + i], bufs[i % 3].at[pl.ds(0, V)], isems[i % 3])

    def out_copy(i):
        return pltpu.make_async_copy(
            bufs[i % 3], out.at[b, s0 + i], osems[i % 3])

    in_copy(0).start()
    for i in range(SPW):
        if i >= 2:
            out_copy(i - 2).wait()
        if i + 1 < SPW:
            in_copy(i + 1).start()
        in_copy(i).wait()
        idx_i = jnp.full((LANE,), i, jnp.int32)
        pg = pv.at[idx_i].get(mode="promise_in_bounds")
        _compute_row(bufs[i % 3], attn16_v, enc_v, pg, i)
        out_copy(i).start()
    for i in range(SPW - 2, SPW):
        out_copy(i).wait()


def kernel(output_logits, attn_weights, decoder_hidden_state, decoder_input,
           context_vector, encoder_input, max_oovs, W_pgen, b_pgen):
    del max_oovs
    w3 = W_pgen.reshape(1, 1, PGEN_D)
    b2 = b_pgen.reshape(1, 1)
    pgen2, aprime = pl.pallas_call(
        _gate_body,
        out_shape=[
            jax.ShapeDtypeStruct((B, S), jnp.float32),
            jax.ShapeDtypeStruct((B, S, ENC), jnp.float32),
        ],
    )(attn_weights, context_vector, decoder_hidden_state, decoder_input,
      w3, b2)

    enc = encoder_input.astype(jnp.int32)
    sc = pl.kernel(
        _sc_body,
        out_type=jax.ShapeDtypeStruct((B, S, VE), jnp.float32),
        mesh=plsc.VectorSubcoreMesh(core_axis_name="c", subcore_axis_name="s"),
        compiler_params=pltpu.CompilerParams(needs_layout_passes=False),
        scratch_types=[
            pltpu.VMEM((VE,), jnp.float32),
            pltpu.VMEM((VE,), jnp.float32),
            pltpu.VMEM((VE,), jnp.float32),
            pltpu.VMEM((SPW, ENC), jnp.float32),
            pltpu.VMEM((ENC,), jnp.int32),
            pltpu.VMEM((SPW,), jnp.float32),
            pltpu.SemaphoreType.DMA,
            pltpu.SemaphoreType.DMA,
            pltpu.SemaphoreType.DMA,
            pltpu.SemaphoreType.DMA,
            pltpu.SemaphoreType.DMA,
            pltpu.SemaphoreType.DMA,
        ],
    )
    total = sc(output_logits, aprime, pgen2, enc)
    return total, pgen2.reshape(B, S, 1)


# speedup vs baseline: 1.1244x; 1.0002x over previous
import jax
import jax.numpy as jnp
from jax import lax
from jax.experimental import pallas as pl
from jax.experimental.pallas import tpu as pltpu
from jax.experimental.pallas import tpu_sc as plsc

B, S, V = 8, 64, 32000
ENC = 400
PGEN_D = 512 + 1024 + 256
OOV = 64
VE = V + OOV
NC, NS = 2, 16
NW = NC * NS
WPB = NW // B
SPW = S // WPB
LANE = 16


EH, DH2, DI = 512, 1024, 256


def _gate_body(attn_ref, ctx_ref, hid_ref, din_ref, w_ref, b_ref,
               pgen_ref, ap_ref):
    w = w_ref[...]
    z = (jnp.sum(ctx_ref[...] * w[:, :, :EH], axis=-1)
         + jnp.sum(hid_ref[...] * w[:, :, EH:EH + DH2], axis=-1)
         + jnp.sum(din_ref[...] * w[:, :, EH + DH2:], axis=-1)
         + b_ref[0, 0])
    pgen = jax.nn.sigmoid(z)
    pgen_ref[...] = pgen[:, :, None]
    ap_ref[...] = attn_ref[...] * (1.0 - pgen)[:, :, None]


def _vsum16(x):
    lane = lax.iota(jnp.int32, LANE)
    for sh in (8, 4, 2, 1):
        idx = jnp.bitwise_xor(lane, sh)
        x = x + x.at[idx].get(mode="promise_in_bounds")
    return x


def _compute_row(row_v, attn16_v, enc_v, pg, i):
    U = 16
    CW = LANE * U

    def pa(j, accs):
        base = j * CW
        a0, a1, a2, a3 = accs
        vs = []
        for k in range(U):
            sl = pl.ds(base + k * LANE, LANE)
            v = jnp.exp(row_v[sl])
            row_v[sl] = v
            vs.append(v)
        for k in range(0, U, 4):
            a0 = a0 + vs[k]
            a1 = a1 + vs[k + 1]
            a2 = a2 + vs[k + 2]
            a3 = a3 + vs[k + 3]
        return (a0, a1, a2, a3)

    zero = jnp.zeros((LANE,), jnp.float32)
    accs = lax.fori_loop(0, V // CW, pa, (zero, zero, zero, zero))
    acc = (accs[0] + accs[1]) + (accs[2] + accs[3])
    t = pg / _vsum16(acc)

    def pb(j, c):
        base = j * CW
        for k in range(U):
            sl = pl.ds(base + k * LANE, LANE)
            row_v[sl] = row_v[sl] * t
        return c

    lax.fori_loop(0, V // CW, pb, 0)
    for j in range(OOV // LANE):
        row_v[pl.ds(V + j * LANE, LANE)] = zero

    for j in range(ENC // LANE):
        sl = pl.ds(j * LANE, LANE)
        plsc.addupdate_scatter(row_v, [enc_v[sl]],
                               attn16_v[i, pl.ds(j * LANE, LANE)])


def _sc_body(logits, aprime, pgen2, enc, out,
             row0_v, row1_v, row2_v, attn16_v, enc_v, pgen_v,
             isem0, isem1, isem2, osem0, osem1, osem2):
    cid = lax.axis_index("c")
    sid = lax.axis_index("s")
    wid = sid * NC + cid
    b = wid // WPB
    s0 = (wid % WPB) * SPW
    pltpu.sync_copy(enc.at[b], enc_v)
    pltpu.sync_copy(pgen2.at[b, pl.ds(s0, SPW)], pgen_v)
    pltpu.sync_copy(aprime.at[b, pl.ds(s0, SPW)], attn16_v)
    pv = pgen_v[...]
    bufs = (row0_v, row1_v, row2_v)
    isems = (isem0, isem1, isem2)
    osems = (osem0, osem1, osem2)

    def in_copy(i):
        return pltpu.make_async_copy(
            logits.at[b, s0 + i], bufs[i % 3].at[pl.ds(0, V)], isems[i % 3])

    def out_copy(i):
        return pltpu.make_async_copy(
            bufs[i % 3], out.at[b, s0 + i], osems[i % 3])

    in_copy(0).start()
    for i in range(SPW):
        if i >= 2:
            out_copy(i - 2).wait()
        if i + 1 < SPW:
            in_copy(i + 1).start()
        in_copy(i).wait()
        idx_i = jnp.full((LANE,), i, jnp.int32)
        pg = pv.at[idx_i].get(mode="promise_in_bounds")
        _compute_row(bufs[i % 3], attn16_v, enc_v, pg, i)
        out_copy(i).start()
    for i in range(SPW - 2, SPW):
        out_copy(i).wait()


def kernel(output_logits, attn_weights, decoder_hidden_state, decoder_input,
           context_vector, encoder_input, max_oovs, W_pgen, b_pgen):
    del max_oovs
    w3 = W_pgen.reshape(1, 1, PGEN_D)
    b2 = b_pgen.reshape(1, 1)
    GB = 2
    pgen2, aprime = pl.pallas_call(
        _gate_body,
        grid=(B // GB,),
        in_specs=[
            pl.BlockSpec((GB, S, ENC), lambda i: (i, 0, 0)),
            pl.BlockSpec((GB, S, EH), lambda i: (i, 0, 0)),
            pl.BlockSpec((GB, S, DH2), lambda i: (i, 0, 0)),
            pl.BlockSpec((GB, S, DI), lambda i: (i, 0, 0)),
            pl.BlockSpec((1, 1, PGEN_D), lambda i: (0, 0, 0)),
            pl.BlockSpec((1, 1), lambda i: (0, 0)),
        ],
        out_specs=[
            pl.BlockSpec((GB, S, 1), lambda i: (i, 0, 0)),
            pl.BlockSpec((GB, S, ENC), lambda i: (i, 0, 0)),
        ],
        out_shape=[
            jax.ShapeDtypeStruct((B, S, 1), jnp.float32),
            jax.ShapeDtypeStruct((B, S, ENC), jnp.float32),
        ],
    )(attn_weights, context_vector, decoder_hidden_state, decoder_input,
      w3, b2)
    pgen3 = pgen2
    pgen2 = pgen3.reshape(B, S)

    enc = encoder_input.astype(jnp.int32)
    sc = pl.kernel(
        _sc_body,
        out_type=jax.ShapeDtypeStruct((B, S, VE), jnp.float32),
        mesh=plsc.VectorSubcoreMesh(core_axis_name="c", subcore_axis_name="s"),
        compiler_params=pltpu.CompilerParams(needs_layout_passes=False),
        scratch_types=[
            pltpu.VMEM((VE,), jnp.float32),
            pltpu.VMEM((VE,), jnp.float32),
            pltpu.VMEM((VE,), jnp.float32),
            pltpu.VMEM((SPW, ENC), jnp.float32),
            pltpu.VMEM((ENC,), jnp.int32),
            pltpu.VMEM((SPW,), jnp.float32),
            pltpu.SemaphoreType.DMA,
            pltpu.SemaphoreType.DMA,
            pltpu.SemaphoreType.DMA,
            pltpu.SemaphoreType.DMA,
            pltpu.SemaphoreType.DMA,
            pltpu.SemaphoreType.DMA,
        ],
    )
    total = sc(output_logits, aprime, pgen2, enc)
    return total, pgen3
